# Initial kernel scaffold; baseline (speedup 1.0000x reference)
import jax
import jax.numpy as jnp
from jax.experimental import pallas as pl


def kernel(x, adj, W1, b1, W2, b2, Wg, a):
    raise NotImplementedError("write your pallas kernel here")


# speedup vs baseline: 1.1468x; 1.1468x over previous
import functools

import jax
import jax.numpy as jnp
from jax.experimental import pallas as pl
from jax.experimental.pallas import tpu as pltpu

N, D, H, Z2 = 4096, 128, 128, 64
BM = 256
NEG = -1000000000000.0


def _leaky(v):
    return jnp.where(v >= 0, v, 0.25 * v)


def _mm_kernel(x_ref, w_ref, o_ref):
    o_ref[:] = jnp.dot(x_ref[:], w_ref[:], preferred_element_type=jnp.float32)


def _mm(x, w):
    m, k = x.shape
    _, n = w.shape
    return pl.pallas_call(
        _mm_kernel,
        out_shape=jax.ShapeDtypeStruct((m, n), jnp.float32),
    )(x, w)


def _gcn_kernel(adj_ref, s_ref, b_ref, o_ref):
    acc = jnp.dot(adj_ref[:], s_ref[:], preferred_element_type=jnp.float32)
    o_ref[:] = _leaky(acc + b_ref[:])


def _gcn(adj, support, b):
    h = support.shape[1]
    return pl.pallas_call(
        _gcn_kernel,
        grid=(N // BM,),
        in_specs=[
            pl.BlockSpec((BM, N), lambda i: (i, 0)),
            pl.BlockSpec((N, h), lambda i: (0, 0)),
            pl.BlockSpec((1, h), lambda i: (0, 0)),
        ],
        out_specs=pl.BlockSpec((BM, h), lambda i: (i, 0)),
        out_shape=jax.ShapeDtypeStruct((N, h), jnp.float32),
        compiler_params=pltpu.CompilerParams(
            dimension_semantics=("parallel",)
        ),
    )(adj, support, b)


def _gat_kernel(adj_ref, h_ref, a1_ref, a2_ref, o_ref):
    i = pl.program_id(0)
    hfull = h_ref[:]
    hb = h_ref[pl.ds(i * BM, BM), :]
    s1 = jnp.sum(hb * a1_ref[:], axis=1, keepdims=True)
    s2 = jnp.sum(hfull * a2_ref[:], axis=1)
    e = _leaky(s1 + s2[None, :])
    e = jnp.where(adj_ref[:] > 0, e, NEG)
    m = jnp.max(e, axis=1, keepdims=True)
    p = jnp.exp(e - m)
    l = jnp.sum(p, axis=1, keepdims=True)
    o = jnp.dot(p, hfull, preferred_element_type=jnp.float32) / l
    o_ref[:] = _leaky(o)


def _gat(adj, h, a1, a2):
    return pl.pallas_call(
        _gat_kernel,
        grid=(N // BM,),
        in_specs=[
            pl.BlockSpec((BM, N), lambda i: (i, 0)),
            pl.BlockSpec((N, Z2), lambda i: (0, 0)),
            pl.BlockSpec((1, Z2), lambda i: (0, 0)),
            pl.BlockSpec((1, Z2), lambda i: (0, 0)),
        ],
        out_specs=pl.BlockSpec((BM, Z2), lambda i: (i, 0)),
        out_shape=jax.ShapeDtypeStruct((N, Z2), jnp.float32),
        compiler_params=pltpu.CompilerParams(
            dimension_semantics=("parallel",)
        ),
    )(adj, h, a1, a2)


def kernel(x, adj, W1, b1, W2, b2, Wg, a):
    b1r = b1.reshape(1, H)
    b2r = b2.reshape(1, H)
    a1r = a[:Z2, 0].reshape(1, Z2)
    a2r = a[Z2:, 0].reshape(1, Z2)
    x1 = _gcn(adj, _mm(x, W1), b1r)
    x2 = _gcn(adj, _mm(x1, W2), b2r)
    h = _mm(x2, Wg)
    out = _gat(adj, h, a1r, a2r)
    return out[:, : Z2 // 2], out[:, Z2 // 2 :]
